# baseline (speedup 1.0000x reference)
import functools

import jax
import jax.numpy as jnp
from jax import lax
from jax.experimental import pallas as pl
from jax.experimental.pallas import tpu as pltpu
from jax.experimental.pallas import tpu_sc as plsc

NUM_CLASSES_K = 1000
BATCH_K = 4096
COLS_K = 20
TOTAL_ROWS_K = COLS_K * NUM_CLASSES_K
NWK = 32
CR_K = 8
NCHUNK_K = TOTAL_ROWS_K // CR_K
CPT_K = -(-NCHUNK_K // NWK)


def _sc_onehot(xt_hbm, out_hbm, xv_ref, buf, sem):
    wid = lax.axis_index("s") * 2 + lax.axis_index("c")
    pltpu.sync_copy(xt_hbm, xv_ref)

    def chunk_body(q, carry):
        ch = wid + NWK * q

        @pl.when(ch < NCHUNK_K)
        def _():
            r0 = ch * CR_K
            t = r0 // NUM_CLASSES_K
            xbase = t * BATCH_K
            for j in range(CR_K):
                c = r0 + j - t * NUM_CLASSES_K

                def inner(k, c2):
                    xv = xv_ref[pl.ds(xbase + k * 16, 16)]
                    buf[j, pl.ds(k * 16, 16)] = jnp.where(
                        xv == c2, jnp.float32(1.0), jnp.float32(0.0)
                    )
                    return c2

                lax.fori_loop(0, BATCH_K // 16, inner, c, unroll=8)
            pltpu.sync_copy(buf, out_hbm.at[pl.ds(r0, CR_K)])

        return carry

    lax.fori_loop(0, CPT_K, chunk_body, 0)


def kernel(x):
    xt = x.astype(jnp.int32).T.reshape(COLS_K * BATCH_K)
    sc = functools.partial(
        pl.kernel,
        out_type=jax.ShapeDtypeStruct((TOTAL_ROWS_K, BATCH_K), jnp.float32),
        mesh=plsc.VectorSubcoreMesh(core_axis_name="c", subcore_axis_name="s"),
        scratch_types=[
            pltpu.VMEM((COLS_K * BATCH_K,), jnp.int32),
            pltpu.VMEM((CR_K, BATCH_K), jnp.float32),
            pltpu.SemaphoreType.DMA,
        ],
    )(_sc_onehot)
    out = sc(xt)
    return out.reshape(COLS_K, NUM_CLASSES_K, BATCH_K).transpose(2, 0, 1)


# speedup vs baseline: 8.5125x; 8.5125x over previous
import jax
import jax.numpy as jnp
from jax.experimental import pallas as pl

NUM_CLASSES_K = 1000
BATCH_K = 4096
COLS_K = 20


def _onehot_body(xt_ref, o_ref):
    t = pl.program_id(0)
    xv = xt_ref[pl.ds(t, 1), :]
    classes = jax.lax.broadcasted_iota(jnp.int32, (NUM_CLASSES_K, BATCH_K), 0)
    o_ref[...] = jnp.where(
        xv == classes, jnp.float32(1.0), jnp.float32(0.0)
    )[None]


def kernel(x):
    xt = x.astype(jnp.int32).T
    out = pl.pallas_call(
        _onehot_body,
        grid=(COLS_K,),
        in_specs=[pl.BlockSpec((COLS_K, BATCH_K), lambda t: (0, 0))],
        out_specs=pl.BlockSpec((1, NUM_CLASSES_K, BATCH_K), lambda t: (t, 0, 0)),
        out_shape=jax.ShapeDtypeStruct((COLS_K, NUM_CLASSES_K, BATCH_K), jnp.float32),
    )(xt)
    return out.transpose(2, 0, 1)


# speedup vs baseline: 8.7556x; 1.0286x over previous
import jax
import jax.numpy as jnp
from jax.experimental import pallas as pl
from jax.experimental.pallas import tpu as pltpu

NUM_CLASSES_K = 1000
BATCH_K = 4096
COLS_K = 20
SUB_K = 200
NSUB_K = NUM_CLASSES_K // SUB_K


def _onehot_body(xt_ref, o_hbm, vbuf, sems):
    t = pl.program_id(0)
    xv = xt_ref[pl.ds(t, 1), :]
    base_iota = jax.lax.broadcasted_iota(jnp.int32, (1, SUB_K, BATCH_K), 1)
    for j in range(NSUB_K):
        n = t * NSUB_K + j
        slot = jax.lax.rem(n, 2)

        @pl.when(n >= 2)
        def _wait():
            pn = n - 2
            pltpu.make_async_copy(
                vbuf.at[slot],
                o_hbm.at[pl.ds(pn // NSUB_K, 1), pl.ds((pn % NSUB_K) * SUB_K, SUB_K), :],
                sems.at[slot],
            ).wait()

        vbuf[slot] = jnp.where(
            xv[None] == base_iota + j * SUB_K, jnp.float32(1.0), jnp.float32(0.0)
        )
        pltpu.make_async_copy(
            vbuf.at[slot],
            o_hbm.at[pl.ds(t, 1), pl.ds(j * SUB_K, SUB_K), :],
            sems.at[slot],
        ).start()

    @pl.when(t == COLS_K - 1)
    def _drain():
        for k in range(2):
            pn = COLS_K * NSUB_K - 2 + k
            pltpu.make_async_copy(
                vbuf.at[jax.lax.rem(jnp.int32(pn), 2)],
                o_hbm.at[pl.ds(pn // NSUB_K, 1), pl.ds((pn % NSUB_K) * SUB_K, SUB_K), :],
                sems.at[jax.lax.rem(jnp.int32(pn), 2)],
            ).wait()


def kernel(x):
    xt = x.astype(jnp.int32).T
    out = pl.pallas_call(
        _onehot_body,
        grid=(COLS_K,),
        in_specs=[pl.BlockSpec((COLS_K, BATCH_K), lambda t: (0, 0))],
        out_specs=pl.BlockSpec(memory_space=pl.ANY),
        out_shape=jax.ShapeDtypeStruct((COLS_K, NUM_CLASSES_K, BATCH_K), jnp.float32),
        scratch_shapes=[
            pltpu.VMEM((2, 1, SUB_K, BATCH_K), jnp.float32),
            pltpu.SemaphoreType.DMA((2,)),
        ],
    )(xt)
    return out.transpose(2, 0, 1)
